# baseline (speedup 1.0000x reference)
import functools

import jax
import jax.numpy as jnp
from jax import lax
from jax.experimental import pallas as pl
from jax.experimental.pallas import tpu as pltpu
from jax.experimental.pallas import tpu_sc as plsc

H = 16
L_Q = 2048
L_K = 2048
EXT = 4096
KR = 16
NBLK = (L_Q // 2) // KR


@functools.partial(
    pl.kernel,
    out_type=jax.ShapeDtypeStruct((1, H, L_Q, L_K), jnp.float32),
    mesh=plsc.VectorSubcoreMesh(core_axis_name="c", subcore_axis_name="s"),
    compiler_params=pltpu.CompilerParams(
        needs_layout_passes=False,
        use_tc_tiling_on_sc=False,
        skip_device_barrier=True,
    ),
    scratch_types=[
        pltpu.VMEM((520,), jnp.float32),
        pltpu.VMEM((16,), jnp.int32),
        pltpu.VMEM((KR, EXT), jnp.float32),
        pltpu.SemaphoreType.DMA,
    ],
)
def _rpe_sc(embT_hbm, dvec_hbm, out_hbm, embrow_v, dvec_v, mat_v, sem):
    c = lax.axis_index("c")
    s = lax.axis_index("s")
    wid = s * 2 + c
    h = wid // 2
    i0 = (wid % 2) * (L_Q // 2)

    pltpu.sync_copy(embT_hbm.at[h], embrow_v)
    pltpu.sync_copy(dvec_hbm, dvec_v)
    vd = dvec_v[...]
    iota = lax.iota(jnp.int32, 16)

    def build(k, carry):
        t = k // (EXT // 16)
        m0 = (k % (EXT // 16)) * 16
        idx = jnp.clip((2296 - m0) + t - iota + vd, 0, 512)
        mat_v[t, pl.ds(m0, 16)] = plsc.load_gather(embrow_v, [idx])
        return carry

    lax.fori_loop(0, KR * (EXT // 16), build, 0)

    def block_refs(b):
        base = i0 + b * KR
        src = mat_v.at[:, pl.ds(pl.multiple_of(2040 - base, 8), L_K)]
        return src, out_hbm.at[0, h, pl.ds(base, KR)]

    def blk(b, carry):
        pltpu.async_copy(*block_refs(b + 1), sem)
        pltpu.make_async_copy(*block_refs(b), sem).wait()
        return carry

    pltpu.async_copy(*block_refs(0), sem)
    lax.fori_loop(0, NBLK - 1, blk, 0)
    pltpu.make_async_copy(*block_refs(NBLK - 1), sem).wait()


def kernel(emb_weight, l_q, l_k):
    embT = jnp.transpose(emb_weight).astype(jnp.float32)
    embT = jnp.pad(embT, ((0, 0), (0, 7)))
    d = jnp.asarray(l_q, jnp.int32) - jnp.asarray(l_k, jnp.int32)
    dvec = jnp.broadcast_to(d, (16,)).astype(jnp.int32)
    return _rpe_sc(embT, dvec)


# speedup vs baseline: 2.5802x; 2.5802x over previous
import functools

import jax
import jax.numpy as jnp
from jax import lax
from jax.experimental import pallas as pl
from jax.experimental.pallas import tpu as pltpu
from jax.experimental.pallas import tpu_sc as plsc

H = 16
L_Q = 2048
L_K = 2048
W = 2944
NCLS = 16
TPC = 8
EMB_PAD = 8320


@functools.partial(
    pl.kernel,
    out_type=jax.ShapeDtypeStruct((1, H, L_Q, L_K), jnp.float32),
    mesh=plsc.VectorSubcoreMesh(core_axis_name="c", subcore_axis_name="s"),
    compiler_params=pltpu.CompilerParams(
        needs_layout_passes=False,
        use_tc_tiling_on_sc=True,
    ),
    scratch_types=[
        pltpu.VMEM((EMB_PAD,), jnp.float32),
        pltpu.VMEM((16,), jnp.int32),
        pltpu.VMEM((8, W), jnp.float32),
        pltpu.VMEM((8, W), jnp.float32),
        pltpu.SemaphoreType.DMA,
    ],
)
def _rpe_sc(emb_hbm, dvec_hbm, out_hbm, emb_v, dvec_v, mat_a, mat_b, sem):
    c = lax.axis_index("c")
    s = lax.axis_index("s")
    wid = s * 2 + c
    h = wid // 2
    i0 = (wid % 2) * (L_Q // 2)

    pltpu.sync_copy(emb_hbm, emb_v)
    pltpu.sync_copy(dvec_hbm, dvec_v)
    vd = dvec_v[...]
    iota = lax.iota(jnp.int32, 16)
    mats = [mat_a, mat_b]

    def band_refs(beta, t, mat):
        src = mat.at[:, pl.ds(pl.multiple_of(128 * (7 - t), 128), L_K)]
        row0 = i0 + 8 * beta + 128 * t
        dst = out_hbm.at[0, h, pl.ds(pl.multiple_of(row0, 8), 8), :]
        return src, dst

    for beta in range(NCLS):
        mat = mats[beta % 2]
        if beta >= 2:
            for t in range(TPC):
                pltpu.make_async_copy(*band_refs(beta - 2, t, mat), sem).wait()
        off_b = 2047 - i0 - 8 * beta - 896
        for r in range(8):
            cb = (2303 + r) - off_b

            def build(k, carry, r=r, cb=cb, mat=mat):
                m0 = k * 16
                idx = jnp.clip((cb - m0) - iota + vd, 0, 512) * 16 + h
                mat[r, pl.ds(m0, 16)] = plsc.load_gather(emb_v, [idx])
                return carry

            lax.fori_loop(0, W // 16, build, 0)
        for t in range(TPC):
            pltpu.async_copy(*band_refs(beta, t, mat), sem)

    for beta in (NCLS - 2, NCLS - 1):
        mat = mats[beta % 2]
        for t in range(TPC):
            pltpu.make_async_copy(*band_refs(beta, t, mat), sem).wait()


def kernel(emb_weight, l_q, l_k):
    emb_flat = jnp.pad(
        emb_weight.astype(jnp.float32).reshape(-1), (0, EMB_PAD - 513 * H)
    )
    d = jnp.asarray(l_q, jnp.int32) - jnp.asarray(l_k, jnp.int32)
    dvec = jnp.broadcast_to(d, (16,)).astype(jnp.int32)
    return _rpe_sc(emb_flat, dvec)
